# Initial kernel scaffold; baseline (speedup 1.0000x reference)
import jax
import jax.numpy as jnp
from jax.experimental import pallas as pl


def kernel(x, edge_index, values, W, b):
    raise NotImplementedError("write your pallas kernel here")


# speedup vs baseline: 5.8273x; 5.8273x over previous
import functools

import jax
import jax.numpy as jnp
from jax import lax
from jax.experimental import pallas as pl
from jax.experimental.pallas import tpu as pltpu
from jax.experimental.pallas import tpu_sc as plsc

N = 10000
E = 320000
D = 128
NC = 2
NS = 16
L = 16

EDGES_PER_TILE = E // (NC * NS)
CHUNK = 80
NCHUNKS = EDGES_PER_TILE // CHUNK
NPAD = 10240
ROWS_PER_TILE = NPAD // NS
ZROWS = 128

_mesh = plsc.VectorSubcoreMesh(
    core_axis_name="c", subcore_axis_name="s", num_cores=NC, num_subcores=NS
)


@functools.partial(
    pl.kernel,
    out_type=jax.ShapeDtypeStruct((NC, NPAD, D), jnp.float32),
    mesh=_mesh,
    scratch_types=[
        pltpu.VMEM_SHARED((NPAD, D), jnp.float32),
        pltpu.VMEM((CHUNK,), jnp.int32),
        pltpu.VMEM((CHUNK,), jnp.int32),
        pltpu.VMEM((CHUNK,), jnp.float32),
        pltpu.VMEM((CHUNK, D), jnp.float32),
        pltpu.VMEM((ZROWS, D), jnp.float32),
        pltpu.SemaphoreType.DMA,
        pltpu.SemaphoreType.DMA,
    ],
)
def _spmm(x_hbm, src_hbm, dst_hbm, val_hbm, p_hbm,
          acc_sh, src_v, dst_v, val_v, rows_v, zrows_v, sem, sem2):
    cid = lax.axis_index("c")
    sid = lax.axis_index("s")

    def zfill(r, carry):
        for c in range(D // L):
            zrows_v[r, pl.ds(c * L, L)] = jnp.zeros((L,), jnp.float32)
        return carry

    lax.fori_loop(0, ZROWS, zfill, 0)
    for i in range(ROWS_PER_TILE // ZROWS):
        pltpu.sync_copy(
            zrows_v, acc_sh.at[pl.ds(sid * ROWS_PER_TILE + i * ZROWS, ZROWS)]
        )
    plsc.subcore_barrier()

    base = (cid * NS + sid) * EDGES_PER_TILE

    def chunk_body(g, carry):
        off = base + g * CHUNK
        c1 = pltpu.async_copy(src_hbm.at[pl.ds(off, CHUNK)], src_v, sem)
        c2 = pltpu.async_copy(dst_hbm.at[pl.ds(off, CHUNK)], dst_v, sem)
        c3 = pltpu.async_copy(val_hbm.at[pl.ds(off, CHUNK)], val_v, sem)
        c1.wait()
        c2.wait()
        c3.wait()
        pltpu.async_copy(x_hbm.at[src_v], rows_v, sem2).wait()

        for g in range(CHUNK // L):
            vv = val_v[pl.ds(g * L, L)]
            for j in range(L):
                bv = jnp.full((L,), vv[j], jnp.float32)
                r = g * L + j
                for c in range(D // L):
                    rows_v[r, pl.ds(c * L, L)] = rows_v[r, pl.ds(c * L, L)] * bv

        pltpu.sync_copy(rows_v, acc_sh.at[dst_v], add=True)
        return carry

    lax.fori_loop(0, NCHUNKS, chunk_body, 0)
    plsc.subcore_barrier()

    for i in range(ROWS_PER_TILE // ZROWS):
        r0 = sid * ROWS_PER_TILE + i * ZROWS
        pltpu.sync_copy(acc_sh.at[pl.ds(r0, ZROWS)], p_hbm.at[cid, pl.ds(r0, ZROWS)])


def _linear_body(p_ref, w_ref, b_ref, o_ref):
    s = p_ref[0] + p_ref[1]
    o_ref[...] = (
        lax.dot_general(
            s,
            w_ref[...],
            (((1,), (1,)), ((), ())),
            preferred_element_type=jnp.float32,
            precision=lax.Precision.HIGHEST,
        )
        + b_ref[...]
    )


_BLK = 2048


def _linear(p, W, b):
    return pl.pallas_call(
        _linear_body,
        grid=(NPAD // _BLK,),
        in_specs=[
            pl.BlockSpec((NC, _BLK, D), lambda i: (0, i, 0)),
            pl.BlockSpec((D, D), lambda i: (0, 0)),
            pl.BlockSpec((1, D), lambda i: (0, 0)),
        ],
        out_specs=pl.BlockSpec((_BLK, D), lambda i: (i, 0)),
        out_shape=jax.ShapeDtypeStruct((NPAD, D), jnp.float32),
    )(p, W, b.reshape(1, D))


def kernel(x, edge_index, values, W, b):
    dst = edge_index[0].astype(jnp.int32)
    src = edge_index[1].astype(jnp.int32)
    p = _spmm(x, src, dst, values)
    return _linear(p, W, b)[:N]


# speedup vs baseline: 6.1087x; 1.0483x over previous
import functools

import jax
import jax.numpy as jnp
from jax import lax
from jax.experimental import pallas as pl
from jax.experimental.pallas import tpu as pltpu
from jax.experimental.pallas import tpu_sc as plsc

N = 10000
E = 320000
D = 128
NC = 2
NS = 16
L = 16

CHUNK = 48
NSLOT = 5
EDGES_PER_TILE = 10080
E_PAD = EDGES_PER_TILE * NC * NS
NCHUNKS = EDGES_PER_TILE // CHUNK
NITER = NCHUNKS // NSLOT
NPAD = 10240
ROWS_PER_TILE = NPAD // NS
ZROWS = 128

_mesh = plsc.VectorSubcoreMesh(
    core_axis_name="c", subcore_axis_name="s", num_cores=NC, num_subcores=NS
)


@functools.partial(
    pl.kernel,
    out_type=jax.ShapeDtypeStruct((NC, NPAD, D), jnp.float32),
    mesh=_mesh,
    scratch_types=[
        pltpu.VMEM_SHARED((NPAD, D), jnp.float32),
        [pltpu.VMEM((CHUNK,), jnp.int32) for _ in range(NSLOT)],
        [pltpu.VMEM((CHUNK,), jnp.int32) for _ in range(NSLOT)],
        [pltpu.VMEM((CHUNK,), jnp.float32) for _ in range(NSLOT)],
        [pltpu.VMEM((CHUNK, D), jnp.float32) for _ in range(NSLOT)],
        pltpu.SemaphoreType.DMA,
        [pltpu.SemaphoreType.DMA for _ in range(NSLOT)],
        pltpu.SemaphoreType.DMA,
    ],
)
def _spmm(x_hbm, src_hbm, dst_hbm, val_hbm, p_hbm,
          acc_sh, src_v, dst_v, val_v, rows_v, sem_i, sem_g, sem_s):
    cid = lax.axis_index("c")
    sid = lax.axis_index("s")

    def zfill(r, carry):
        for c in range(D // L):
            rows_v[0][r, pl.ds(c * L, L)] = jnp.zeros((L,), jnp.float32)
        return carry

    lax.fori_loop(0, CHUNK, zfill, 0)
    for i in range(ROWS_PER_TILE // CHUNK):
        pltpu.sync_copy(
            rows_v[0], acc_sh.at[pl.ds(sid * ROWS_PER_TILE + i * CHUNK, CHUNK)]
        )
    _ztail = ROWS_PER_TILE % CHUNK
    if _ztail:
        pltpu.sync_copy(
            rows_v[0].at[pl.ds(0, _ztail)],
            acc_sh.at[pl.ds(sid * ROWS_PER_TILE + ROWS_PER_TILE - _ztail, _ztail)],
        )
    plsc.subcore_barrier()

    base = (cid * NS + sid) * EDGES_PER_TILE

    def iter_body(it, carry):
        goff = base + it * NSLOT * CHUNK
        for k in range(NSLOT):
            off = goff + k * CHUNK
            pltpu.async_copy(src_hbm.at[pl.ds(off, CHUNK)], src_v[k], sem_i)
            pltpu.async_copy(val_hbm.at[pl.ds(off, CHUNK)], val_v[k], sem_i)

        @pl.when(it != 0)
        def _():
            for k in range(NSLOT):
                pltpu.make_async_copy(rows_v[k], acc_sh.at[dst_v[k]], sem_s).wait()

        for k in range(NSLOT):
            off = goff + k * CHUNK
            pltpu.async_copy(dst_hbm.at[pl.ds(off, CHUNK)], dst_v[k], sem_i)

        for k in range(NSLOT):
            pltpu.make_async_copy(src_hbm.at[pl.ds(goff, CHUNK)], src_v[k], sem_i).wait()
            pltpu.make_async_copy(val_hbm.at[pl.ds(goff, CHUNK)], val_v[k], sem_i).wait()
            pltpu.make_async_copy(dst_hbm.at[pl.ds(goff, CHUNK)], dst_v[k], sem_i).wait()
        gathers = []
        for k in range(NSLOT):
            gathers.append(pltpu.async_copy(x_hbm.at[src_v[k]], rows_v[k], sem_g[k]))

        for k in range(NSLOT):
            gathers[k].wait()
            for g in range(CHUNK // L):
                vv = val_v[k][pl.ds(g * L, L)]
                for j in range(L):
                    bv = jnp.full((L,), vv[j], jnp.float32)
                    r = g * L + j
                    for c in range(D // L):
                        rows_v[k][r, pl.ds(c * L, L)] = (
                            rows_v[k][r, pl.ds(c * L, L)] * bv
                        )
            pltpu.async_copy(rows_v[k], acc_sh.at[dst_v[k]], sem_s, add=True)
        return carry

    lax.fori_loop(0, NITER, iter_body, 0)
    for k in range(NSLOT):
        pltpu.make_async_copy(rows_v[k], acc_sh.at[dst_v[k]], sem_s).wait()
    plsc.subcore_barrier()

    for i in range(ROWS_PER_TILE // ZROWS):
        r0 = sid * ROWS_PER_TILE + i * ZROWS
        pltpu.sync_copy(acc_sh.at[pl.ds(r0, ZROWS)], p_hbm.at[cid, pl.ds(r0, ZROWS)])


def _linear_body(p_ref, w_ref, b_ref, o_ref):
    s = p_ref[0] + p_ref[1]
    o_ref[...] = (
        lax.dot_general(
            s,
            w_ref[...],
            (((1,), (1,)), ((), ())),
            preferred_element_type=jnp.float32,
            precision=lax.Precision.HIGHEST,
        )
        + b_ref[...]
    )


_BLK = 2048


def _linear(p, W, b):
    return pl.pallas_call(
        _linear_body,
        grid=(NPAD // _BLK,),
        in_specs=[
            pl.BlockSpec((NC, _BLK, D), lambda i: (0, i, 0)),
            pl.BlockSpec((D, D), lambda i: (0, 0)),
            pl.BlockSpec((1, D), lambda i: (0, 0)),
        ],
        out_specs=pl.BlockSpec((_BLK, D), lambda i: (i, 0)),
        out_shape=jax.ShapeDtypeStruct((NPAD, D), jnp.float32),
    )(p, W, b.reshape(1, D))


def kernel(x, edge_index, values, W, b):
    dst = edge_index[0].astype(jnp.int32)
    src = edge_index[1].astype(jnp.int32)
    pad = E_PAD - E
    zi = jnp.zeros((pad,), jnp.int32)
    dst = jnp.concatenate([dst, zi])
    src = jnp.concatenate([src, zi])
    vals = jnp.concatenate([values, jnp.zeros((pad,), jnp.float32)])
    p = _spmm(x, src, dst, vals)
    return _linear(p, W, b)[:N]


# speedup vs baseline: 8.4267x; 1.3795x over previous
import functools

import jax
import jax.numpy as jnp
from jax import lax
from jax.experimental import pallas as pl
from jax.experimental.pallas import tpu as pltpu
from jax.experimental.pallas import tpu_sc as plsc

N = 10000
E = 320000
D = 128
NC = 2
NS = 16
L = 16
DH = D // NC

CHUNK = 56
NSLOT = 5
EDGES_PER_TILE = 20160
E_PAD = EDGES_PER_TILE * NS
NCHUNKS = EDGES_PER_TILE // CHUNK
NITER = NCHUNKS // NSLOT
NPAD = 10240
ROWS_PER_TILE = NPAD // NS

_mesh = plsc.VectorSubcoreMesh(
    core_axis_name="c", subcore_axis_name="s", num_cores=NC, num_subcores=NS
)


@functools.partial(
    pl.kernel,
    out_type=jax.ShapeDtypeStruct((NC, NPAD, DH), jnp.float32),
    mesh=_mesh,
    scratch_types=[
        pltpu.VMEM_SHARED((NPAD, DH), jnp.float32),
        pltpu.VMEM_SHARED((NPAD, DH), jnp.float32),
        [pltpu.VMEM((CHUNK,), jnp.int32) for _ in range(NSLOT)],
        [pltpu.VMEM((CHUNK,), jnp.int32) for _ in range(NSLOT)],
        [pltpu.VMEM((CHUNK + 8,), jnp.float32) for _ in range(NSLOT)],
        [pltpu.VMEM((CHUNK, DH), jnp.float32) for _ in range(NSLOT)],
        pltpu.SemaphoreType.DMA,
        [pltpu.SemaphoreType.DMA for _ in range(NSLOT)],
        pltpu.SemaphoreType.DMA,
        pltpu.SemaphoreType.DMA,
    ],
)
def _spmm(x_hbm, src_hbm, dst_hbm, val_hbm, p_hbm,
          xsh, acc_sh, src_v, dst_v, val_v, rows_v, sem_i, sem_g, sem_s, sem_x):
    cid = lax.axis_index("c")
    sid = lax.axis_index("s")
    r0 = sid * ROWS_PER_TILE

    xcp = pltpu.async_copy(
        x_hbm.at[cid, pl.ds(r0, ROWS_PER_TILE)],
        xsh.at[pl.ds(r0, ROWS_PER_TILE)],
        sem_x,
    )

    def zfill(r, carry):
        for c in range(DH // L):
            rows_v[0][r, pl.ds(c * L, L)] = jnp.zeros((L,), jnp.float32)
        return carry

    lax.fori_loop(0, CHUNK, zfill, 0)
    for i in range(ROWS_PER_TILE // CHUNK):
        pltpu.sync_copy(rows_v[0], acc_sh.at[pl.ds(r0 + i * CHUNK, CHUNK)])
    _zt = ROWS_PER_TILE % CHUNK
    if _zt:
        pltpu.sync_copy(
            rows_v[0].at[pl.ds(0, _zt)],
            acc_sh.at[pl.ds(r0 + ROWS_PER_TILE - _zt, _zt)],
        )
    xcp.wait()
    plsc.subcore_barrier()

    base = sid * EDGES_PER_TILE

    def iter_body(it, carry):
        goff = base + it * NSLOT * CHUNK
        for k in range(NSLOT):
            off = goff + k * CHUNK
            pltpu.async_copy(src_hbm.at[pl.ds(off, CHUNK)], src_v[k], sem_i)
            pltpu.async_copy(val_hbm.at[pl.ds(off, CHUNK)], val_v[k].at[pl.ds(0, CHUNK)], sem_i)

        @pl.when(it != 0)
        def _():
            for k in range(NSLOT):
                pltpu.make_async_copy(rows_v[k], acc_sh.at[dst_v[k]], sem_s).wait()

        for k in range(NSLOT):
            off = goff + k * CHUNK
            pltpu.async_copy(dst_hbm.at[pl.ds(off, CHUNK)], dst_v[k], sem_i)

        for k in range(NSLOT):
            pltpu.make_async_copy(src_hbm.at[pl.ds(goff, CHUNK)], src_v[k], sem_i).wait()
            pltpu.make_async_copy(val_hbm.at[pl.ds(goff, CHUNK)], val_v[k].at[pl.ds(0, CHUNK)], sem_i).wait()
            pltpu.make_async_copy(dst_hbm.at[pl.ds(goff, CHUNK)], dst_v[k], sem_i).wait()
        gathers = []
        for k in range(NSLOT):
            gathers.append(pltpu.async_copy(xsh.at[src_v[k]], rows_v[k], sem_g[k]))

        for k in range(NSLOT):
            gathers[k].wait()
            for g0 in range(0, CHUNK, L):
                vv = val_v[k][pl.ds(g0, L)]
                for j in range(min(L, CHUNK - g0)):
                    bv = jnp.full((L,), vv[j], jnp.float32)
                    r = g0 + j
                    for c in range(DH // L):
                        rows_v[k][r, pl.ds(c * L, L)] = (
                            rows_v[k][r, pl.ds(c * L, L)] * bv
                        )
            pltpu.async_copy(rows_v[k], acc_sh.at[dst_v[k]], sem_s, add=True)
        return carry

    lax.fori_loop(0, NITER, iter_body, 0)
    for k in range(NSLOT):
        pltpu.make_async_copy(rows_v[k], acc_sh.at[dst_v[k]], sem_s).wait()
    plsc.subcore_barrier()

    pltpu.sync_copy(
        acc_sh.at[pl.ds(r0, ROWS_PER_TILE)],
        p_hbm.at[cid, pl.ds(r0, ROWS_PER_TILE)],
    )


def _linear_body(p_ref, w_ref, b_ref, o_ref):
    s = jnp.concatenate([p_ref[0], p_ref[1]], axis=1)
    o_ref[...] = (
        lax.dot_general(
            s,
            w_ref[...],
            (((1,), (1,)), ((), ())),
            preferred_element_type=jnp.float32,
            precision=lax.Precision.HIGHEST,
        )
        + b_ref[...]
    )


_BLK = 2048


def _linear(p, W, b):
    return pl.pallas_call(
        _linear_body,
        grid=(NPAD // _BLK,),
        in_specs=[
            pl.BlockSpec((NC, _BLK, DH), lambda i: (0, i, 0)),
            pl.BlockSpec((D, D), lambda i: (0, 0)),
            pl.BlockSpec((1, D), lambda i: (0, 0)),
        ],
        out_specs=pl.BlockSpec((_BLK, D), lambda i: (i, 0)),
        out_shape=jax.ShapeDtypeStruct((NPAD, D), jnp.float32),
    )(p, W, b.reshape(1, D))


def kernel(x, edge_index, values, W, b):
    dst = edge_index[0].astype(jnp.int32)
    src = edge_index[1].astype(jnp.int32)
    pad = E_PAD - E
    zi = jnp.zeros((pad,), jnp.int32)
    dst = jnp.concatenate([dst, zi])
    src = jnp.concatenate([src, zi])
    vals = jnp.concatenate([values, jnp.zeros((pad,), jnp.float32)])
    xp = jnp.concatenate([x, jnp.zeros((NPAD - N, D), x.dtype)], axis=0)
    xr = jnp.stack([xp[:, :DH], xp[:, DH:]])
    p = _spmm(xr, src, dst, vals)
    return _linear(p, W, b)[:N]


# speedup vs baseline: 8.5710x; 1.0171x over previous
import functools

import jax
import jax.numpy as jnp
from jax import lax
from jax.experimental import pallas as pl
from jax.experimental.pallas import tpu as pltpu
from jax.experimental.pallas import tpu_sc as plsc

N = 10000
E = 320000
D = 128
NC = 2
NS = 16
L = 16
DH = D // NC

CHUNK = 56
NSLOT = 5
EDGES_PER_TILE = 20160
E_PAD = EDGES_PER_TILE * NS
NCHUNKS = EDGES_PER_TILE // CHUNK
NITER = NCHUNKS // NSLOT
NPAD = 10240
ROWS_PER_TILE = NPAD // NS

_mesh = plsc.VectorSubcoreMesh(
    core_axis_name="c", subcore_axis_name="s", num_cores=NC, num_subcores=NS
)


@functools.partial(
    pl.kernel,
    out_type=jax.ShapeDtypeStruct((NC, NPAD, DH), jnp.float32),
    mesh=_mesh,
    scratch_types=[
        pltpu.VMEM_SHARED((NPAD, DH), jnp.float32),
        pltpu.VMEM_SHARED((NPAD, DH), jnp.float32),
        [pltpu.VMEM((CHUNK,), jnp.int32) for _ in range(NSLOT)],
        [pltpu.VMEM((CHUNK,), jnp.int32) for _ in range(NSLOT)],
        [pltpu.VMEM((CHUNK + 8,), jnp.float32) for _ in range(NSLOT)],
        [pltpu.VMEM((CHUNK, DH), jnp.float32) for _ in range(NSLOT)],
        pltpu.SemaphoreType.DMA,
        [pltpu.SemaphoreType.DMA for _ in range(NSLOT)],
        pltpu.SemaphoreType.DMA,
        pltpu.SemaphoreType.DMA,
    ],
)
def _spmm(x_hbm, src_hbm, dst_hbm, val_hbm, p_hbm,
          xsh, acc_sh, src_v, dst_v, val_v, rows_v, sem_i, sem_g, sem_s, sem_x):
    cid = lax.axis_index("c")
    sid = lax.axis_index("s")
    r0 = sid * ROWS_PER_TILE

    xcp = pltpu.async_copy(
        x_hbm.at[cid, pl.ds(r0, ROWS_PER_TILE)],
        xsh.at[pl.ds(r0, ROWS_PER_TILE)],
        sem_x,
    )

    def zfill(r, carry):
        for c in range(DH // L):
            rows_v[0][r, pl.ds(c * L, L)] = jnp.zeros((L,), jnp.float32)
        return carry

    lax.fori_loop(0, CHUNK, zfill, 0)
    zcps = [
        pltpu.async_copy(rows_v[0], acc_sh.at[pl.ds(r0 + i * CHUNK, CHUNK)], sem_s)
        for i in range(ROWS_PER_TILE // CHUNK)
    ]
    _zt = ROWS_PER_TILE % CHUNK
    if _zt:
        zcps.append(
            pltpu.async_copy(
                rows_v[0].at[pl.ds(0, _zt)],
                acc_sh.at[pl.ds(r0 + ROWS_PER_TILE - _zt, _zt)],
                sem_s,
            )
        )
    for cp in zcps:
        cp.wait()
    xcp.wait()
    plsc.subcore_barrier()

    base = sid * EDGES_PER_TILE

    def iter_body(it, carry):
        goff = base + it * NSLOT * CHUNK
        for k in range(NSLOT):
            off = goff + k * CHUNK
            pltpu.async_copy(src_hbm.at[pl.ds(off, CHUNK)], src_v[k], sem_i)
            pltpu.async_copy(val_hbm.at[pl.ds(off, CHUNK)], val_v[k].at[pl.ds(0, CHUNK)], sem_i)

        @pl.when(it != 0)
        def _():
            for k in range(NSLOT):
                pltpu.make_async_copy(rows_v[k], acc_sh.at[dst_v[k]], sem_s).wait()

        for k in range(NSLOT):
            off = goff + k * CHUNK
            pltpu.async_copy(dst_hbm.at[pl.ds(off, CHUNK)], dst_v[k], sem_i)

        for k in range(NSLOT):
            pltpu.make_async_copy(src_hbm.at[pl.ds(goff, CHUNK)], src_v[k], sem_i).wait()
            pltpu.make_async_copy(val_hbm.at[pl.ds(goff, CHUNK)], val_v[k].at[pl.ds(0, CHUNK)], sem_i).wait()
            pltpu.make_async_copy(dst_hbm.at[pl.ds(goff, CHUNK)], dst_v[k], sem_i).wait()
        gathers = []
        for k in range(NSLOT):
            gathers.append(pltpu.async_copy(xsh.at[src_v[k]], rows_v[k], sem_g[k]))

        for k in range(NSLOT):
            gathers[k].wait()
            for g0 in range(0, CHUNK, L):
                vv = val_v[k][pl.ds(g0, L)]
                for j in range(min(L, CHUNK - g0)):
                    bv = jnp.full((L,), vv[j], jnp.float32)
                    r = g0 + j
                    for c in range(DH // L):
                        rows_v[k][r, pl.ds(c * L, L)] = (
                            rows_v[k][r, pl.ds(c * L, L)] * bv
                        )
            pltpu.async_copy(rows_v[k], acc_sh.at[dst_v[k]], sem_s, add=True)
        return carry

    lax.fori_loop(0, NITER, iter_body, 0)
    for k in range(NSLOT):
        pltpu.make_async_copy(rows_v[k], acc_sh.at[dst_v[k]], sem_s).wait()
    plsc.subcore_barrier()

    pltpu.sync_copy(
        acc_sh.at[pl.ds(r0, ROWS_PER_TILE)],
        p_hbm.at[cid, pl.ds(r0, ROWS_PER_TILE)],
    )


def _linear_body(p_ref, w_ref, b_ref, o_ref):
    s = jnp.concatenate([p_ref[0], p_ref[1]], axis=1)
    o_ref[...] = (
        lax.dot_general(
            s,
            w_ref[...],
            (((1,), (1,)), ((), ())),
            preferred_element_type=jnp.float32,
            precision=lax.Precision.HIGHEST,
        )
        + b_ref[...]
    )


_BLK = 2000


def _linear(p, W, b):
    return pl.pallas_call(
        _linear_body,
        grid=(N // _BLK,),
        in_specs=[
            pl.BlockSpec((NC, _BLK, DH), lambda i: (0, i, 0)),
            pl.BlockSpec((D, D), lambda i: (0, 0)),
            pl.BlockSpec((1, D), lambda i: (0, 0)),
        ],
        out_specs=pl.BlockSpec((_BLK, D), lambda i: (i, 0)),
        out_shape=jax.ShapeDtypeStruct((N, D), jnp.float32),
    )(p, W, b.reshape(1, D))


def kernel(x, edge_index, values, W, b):
    dst = edge_index[0].astype(jnp.int32)
    src = edge_index[1].astype(jnp.int32)
    pad = E_PAD - E
    zi = jnp.zeros((pad,), jnp.int32)
    dst = jnp.concatenate([dst, zi])
    src = jnp.concatenate([src, zi])
    vals = jnp.concatenate([values, jnp.zeros((pad,), jnp.float32)])
    xp = jnp.concatenate([x, jnp.zeros((NPAD - N, D), x.dtype)], axis=0)
    xr = jnp.stack([xp[:, :DH], xp[:, DH:]])
    p = _spmm(xr, src, dst, vals)
    return _linear(p, W, b)
